# baseline (speedup 1.0000x reference)
import functools

import jax
import jax.numpy as jnp
from jax import lax
from jax.experimental import pallas as pl
from jax.experimental.pallas import tpu as pltpu
from jax.experimental.pallas import tpu_sc as plsc

REG_WEIGHT_ = 0.01
NC = 2
NS = 16
NW = NC * NS
CW = 128
BIG_CORE = 0
BIG_FRAC = 0.507


def _tables_body(x_ref, w_ref, b_ref, p_ref, q_ref):
    d = x_ref.shape[1]
    x = x_ref[...]
    dims = (((0,), (1,)), ((), ()))
    p_ref[...] = (lax.dot_general(w_ref[:d, :], x, dims,
                                  preferred_element_type=jnp.float32)
                  + b_ref[...])
    q_ref[...] = lax.dot_general(w_ref[d:, :], x, dims,
                                 preferred_element_type=jnp.float32)


def _combine_body(n_nodes, n_edges, y_ref, t_ref, part_ref, o_ref):
    base = jnp.sum(jnp.abs(y_ref[...] - t_ref[...]))
    l1 = jnp.sum(part_ref[...])
    total = base / n_nodes + REG_WEIGHT_ * (l1 / n_edges)
    o_ref[...] = jnp.reshape(total, (1, 1))


def _make_edge_l1(n_edges, msg_dim):
    total_chunks = n_edges // CW
    nch_big = int(total_chunks * BIG_FRAC / NS) // 2 * 2
    small_total = total_chunks - NS * nch_big
    nch_small = small_total // NS // 2 * 2
    extra2 = (small_total - NS * nch_small) // 2
    assert nch_small * NS + 2 * extra2 == small_total and extra2 <= NS
    stage_max = max(nch_big, nch_small + (2 if extra2 else 0)) * CW

    mesh = plsc.VectorSubcoreMesh(core_axis_name="c", subcore_axis_name="s")

    @functools.partial(
        pl.kernel,
        mesh=mesh,
        out_type=jax.ShapeDtypeStruct((NW, msg_dim), jnp.float32),
        compiler_params=pltpu.CompilerParams(use_tc_tiling_on_sc=False),
        scratch_types=[
            pltpu.VMEM((stage_max,), jnp.int32),
            pltpu.VMEM((stage_max,), jnp.int32),
            pltpu.VMEM((CW, msg_dim), jnp.float32),
            pltpu.VMEM((CW, msg_dim), jnp.float32),
            pltpu.VMEM((CW, msg_dim), jnp.float32),
            pltpu.VMEM((CW, msg_dim), jnp.float32),
            pltpu.VMEM((msg_dim,), jnp.float32),
            pltpu.SemaphoreType.DMA,
            pltpu.SemaphoreType.DMA,
            pltpu.SemaphoreType.DMA,
            pltpu.SemaphoreType.DMA,
        ],
    )
    def edge_l1(p_hbm, q_hbm, ei_hbm, out_hbm,
                sidx, didx, pbuf0, qbuf0, pbuf1, qbuf1, accv,
                sem_p0, sem_q0, sem_p1, sem_q1):
        c = lax.axis_index("c")
        s = lax.axis_index("s")
        wid = s * NC + c
        is_big = c == BIG_CORE
        nc_mine = jnp.where(is_big, nch_big,
                            jnp.where(s < extra2, nch_small + 2, nch_small))
        start_chunk = jnp.where(
            is_big, small_total + s * nch_big,
            s * nch_small + 2 * jnp.minimum(s, extra2))
        e0 = start_chunk * CW
        stage_start = jnp.minimum(e0, n_edges - stage_max)
        off = e0 - stage_start
        pltpu.sync_copy(ei_hbm.at[0, pl.ds(stage_start, stage_max)], sidx)
        pltpu.sync_copy(ei_hbm.at[1, pl.ds(stage_start, stage_max)], didx)

        def issue(k, pb, qb, sp, sq):
            pltpu.async_copy(p_hbm.at[sidx.at[pl.ds(off + k * CW, CW)]], pb, sp)
            pltpu.async_copy(q_hbm.at[didx.at[pl.ds(off + k * CW, CW)]], qb, sq)

        def drain(k, pb, qb, sp, sq):
            pltpu.make_async_copy(p_hbm.at[sidx.at[pl.ds(off + k * CW, CW)]], pb, sp).wait()
            pltpu.make_async_copy(q_hbm.at[didx.at[pl.ds(off + k * CW, CW)]], qb, sq).wait()

        def accum(pb, qb, acc):
            def lane_body(i, carry):
                a0, a1 = carry
                j = i * 2
                a0 = a0 + jnp.abs(pb[j] + qb[j])
                a1 = a1 + jnp.abs(pb[j + 1] + qb[j + 1])
                return a0, a1

            return lax.fori_loop(0, CW // 2, lane_body, acc, unroll=4)

        issue(0, pbuf0, qbuf0, sem_p0, sem_q0)
        zero = jnp.zeros((msg_dim,), jnp.float32)

        def pair_body(h, acc):
            k = h * 2
            issue(k + 1, pbuf1, qbuf1, sem_p1, sem_q1)
            drain(k, pbuf0, qbuf0, sem_p0, sem_q0)
            acc = accum(pbuf0, qbuf0, acc)

            @pl.when(k + 2 < nc_mine)
            def _():
                issue(k + 2, pbuf0, qbuf0, sem_p0, sem_q0)

            drain(k + 1, pbuf1, qbuf1, sem_p1, sem_q1)
            return accum(pbuf1, qbuf1, acc)

        a0, a1 = lax.fori_loop(0, nc_mine // 2, pair_body, (zero, zero))
        accv[...] = a0 + a1
        pltpu.sync_copy(accv, out_hbm.at[wid])

    return edge_l1


def kernel(y, target, x, edge_index, W_msg, b_msg):
    n_nodes, d_feat = x.shape
    n_edges = edge_index.shape[1]
    msg_dim = W_msg.shape[1]

    ei = edge_index.astype(jnp.int32)
    b2 = b_msg.reshape(msg_dim, 1)

    tables = pl.pallas_call(
        _tables_body,
        out_shape=(jax.ShapeDtypeStruct((msg_dim, n_nodes), jnp.float32),
                   jax.ShapeDtypeStruct((msg_dim, n_nodes), jnp.float32)),
    )
    p_t, q_t = tables(x, W_msg, b2)
    p_tab = p_t.T
    q_tab = q_t.T

    partials = _make_edge_l1(n_edges, msg_dim)(p_tab, q_tab, ei)

    y2 = y.reshape(80, -1)
    t2 = target.reshape(80, -1)
    combine = pl.pallas_call(
        functools.partial(_combine_body, n_nodes, n_edges),
        out_shape=jax.ShapeDtypeStruct((1, 1), jnp.float32),
    )
    return combine(y2, t2, partials)[0, 0]


# speedup vs baseline: 1.0373x; 1.0373x over previous
import functools

import jax
import jax.numpy as jnp
from jax import lax
from jax.experimental import pallas as pl
from jax.experimental.pallas import tpu as pltpu
from jax.experimental.pallas import tpu_sc as plsc

REG_WEIGHT_ = 0.01
NC = 2
NS = 16
NW = NC * NS
CW = 128
BIG_CORE = 0
BIG_FRAC = 0.507


def _tables_body(x_ref, w_ref, b_ref, p_ref, q_ref):
    d = x_ref.shape[1]
    x = x_ref[...]
    p_ref[...] = (jnp.dot(x, w_ref[:d, :], preferred_element_type=jnp.float32)
                  + b_ref[...])
    q_ref[...] = jnp.dot(x, w_ref[d:, :], preferred_element_type=jnp.float32)


def _combine_body(n_nodes, n_edges, y_ref, t_ref, part_ref, o_ref):
    base = jnp.sum(jnp.abs(y_ref[...] - t_ref[...]))
    l1 = jnp.sum(part_ref[...])
    total = base / n_nodes + REG_WEIGHT_ * (l1 / n_edges)
    o_ref[...] = jnp.reshape(total, (1, 1))


def _make_edge_l1(n_edges, msg_dim):
    total_chunks = n_edges // CW
    nch_big = int(total_chunks * BIG_FRAC / NS) // 2 * 2
    small_total = total_chunks - NS * nch_big
    nch_small = small_total // NS // 2 * 2
    extra2 = (small_total - NS * nch_small) // 2
    assert nch_small * NS + 2 * extra2 == small_total and extra2 <= NS
    stage_max = max(nch_big, nch_small + (2 if extra2 else 0)) * CW

    mesh = plsc.VectorSubcoreMesh(core_axis_name="c", subcore_axis_name="s")
    assert (NW * msg_dim) % 128 == 0

    @functools.partial(
        pl.kernel,
        mesh=mesh,
        out_type=jax.ShapeDtypeStruct((NW * msg_dim // 128, 128), jnp.float32),
        compiler_params=pltpu.CompilerParams(use_tc_tiling_on_sc=False),
        scratch_types=[
            pltpu.VMEM((stage_max,), jnp.int32),
            pltpu.VMEM((stage_max,), jnp.int32),
            pltpu.VMEM((CW, msg_dim), jnp.float32),
            pltpu.VMEM((CW, msg_dim), jnp.float32),
            pltpu.VMEM((CW, msg_dim), jnp.float32),
            pltpu.VMEM((CW, msg_dim), jnp.float32),
            pltpu.VMEM((msg_dim,), jnp.float32),
            pltpu.SemaphoreType.DMA,
            pltpu.SemaphoreType.DMA,
            pltpu.SemaphoreType.DMA,
            pltpu.SemaphoreType.DMA,
        ],
    )
    def edge_l1(p_hbm, q_hbm, ei_hbm, out_hbm,
                sidx, didx, pbuf0, qbuf0, pbuf1, qbuf1, accv,
                sem_p0, sem_q0, sem_p1, sem_q1):
        c = lax.axis_index("c")
        s = lax.axis_index("s")
        wid = s * NC + c
        is_big = c == BIG_CORE
        nc_mine = jnp.where(is_big, nch_big,
                            jnp.where(s < extra2, nch_small + 2, nch_small))
        start_chunk = jnp.where(
            is_big, small_total + s * nch_big,
            s * nch_small + 2 * jnp.minimum(s, extra2))
        e0 = start_chunk * CW
        stage_start = jnp.minimum(e0, n_edges - stage_max)
        off = e0 - stage_start
        pltpu.sync_copy(ei_hbm.at[0, pl.ds(stage_start, stage_max)], sidx)
        pltpu.sync_copy(ei_hbm.at[1, pl.ds(stage_start, stage_max)], didx)

        def issue(k, pb, qb, sp, sq):
            pltpu.async_copy(p_hbm.at[sidx.at[pl.ds(off + k * CW, CW)]], pb, sp)
            pltpu.async_copy(q_hbm.at[didx.at[pl.ds(off + k * CW, CW)]], qb, sq)

        def drain(k, pb, qb, sp, sq):
            pltpu.make_async_copy(p_hbm.at[sidx.at[pl.ds(off + k * CW, CW)]], pb, sp).wait()
            pltpu.make_async_copy(q_hbm.at[didx.at[pl.ds(off + k * CW, CW)]], qb, sq).wait()

        def accum(pb, qb, acc):
            def lane_body(i, carry):
                a0, a1 = carry
                j = i * 2
                a0 = a0 + jnp.abs(pb[j] + qb[j])
                a1 = a1 + jnp.abs(pb[j + 1] + qb[j + 1])
                return a0, a1

            return lax.fori_loop(0, CW // 2, lane_body, acc, unroll=4)

        issue(0, pbuf0, qbuf0, sem_p0, sem_q0)
        zero = jnp.zeros((msg_dim,), jnp.float32)

        def pair_body(h, acc):
            k = h * 2
            issue(k + 1, pbuf1, qbuf1, sem_p1, sem_q1)
            drain(k, pbuf0, qbuf0, sem_p0, sem_q0)
            acc = accum(pbuf0, qbuf0, acc)

            @pl.when(k + 2 < nc_mine)
            def _():
                issue(k + 2, pbuf0, qbuf0, sem_p0, sem_q0)

            drain(k + 1, pbuf1, qbuf1, sem_p1, sem_q1)
            return accum(pbuf1, qbuf1, acc)

        a0, a1 = lax.fori_loop(0, nc_mine // 2, pair_body, (zero, zero))
        accv[...] = a0 + a1
        per_row = 128 // msg_dim
        pltpu.sync_copy(
            accv, out_hbm.at[wid // per_row, pl.ds((wid % per_row) * msg_dim, msg_dim)])

    return edge_l1


def kernel(y, target, x, edge_index, W_msg, b_msg):
    n_nodes, d_feat = x.shape
    n_edges = edge_index.shape[1]
    msg_dim = W_msg.shape[1]

    ei = edge_index.astype(jnp.int32)
    b2 = b_msg.reshape(1, msg_dim)

    tables = pl.pallas_call(
        _tables_body,
        out_shape=(jax.ShapeDtypeStruct((n_nodes, msg_dim), jnp.float32),
                   jax.ShapeDtypeStruct((n_nodes, msg_dim), jnp.float32)),
    )
    p_tab, q_tab = tables(x, W_msg, b2)

    partials = _make_edge_l1(n_edges, msg_dim)(p_tab, q_tab, ei)

    y2 = y.reshape(80, -1)
    t2 = target.reshape(80, -1)
    combine = pl.pallas_call(
        functools.partial(_combine_body, n_nodes, n_edges),
        out_shape=jax.ShapeDtypeStruct((1, 1), jnp.float32),
    )
    return combine(y2, t2, partials)[0, 0]


# speedup vs baseline: 1.0745x; 1.0359x over previous
import functools

import jax
import jax.numpy as jnp
from jax import lax
from jax.experimental import pallas as pl
from jax.experimental.pallas import tpu as pltpu
from jax.experimental.pallas import tpu_sc as plsc

REG_WEIGHT_ = 0.01
NC = 2
NS = 16
NW = NC * NS
CW = 128
BIG_CORE = 0
BIG_FRAC = 0.507


def _tables_body(x_ref, w_ref, b_ref, pq_ref):
    d = x_ref.shape[1]
    x = x_ref[...]
    p = jnp.dot(x, w_ref[:d, :], preferred_element_type=jnp.float32) + b_ref[...]
    q = jnp.dot(x, w_ref[d:, :], preferred_element_type=jnp.float32)
    pq_ref[...] = jnp.concatenate([p, q], axis=1)


def _combine_body(n_nodes, n_edges, y_ref, t_ref, part_ref, o_ref):
    base = jnp.sum(jnp.abs(y_ref[...] - t_ref[...]))
    l1 = jnp.sum(part_ref[...])
    total = base / n_nodes + REG_WEIGHT_ * (l1 / n_edges)
    o_ref[...] = jnp.reshape(total, (1, 1))


def _make_edge_l1(n_edges, msg_dim):
    total_chunks = n_edges // CW
    nch_big = int(total_chunks * BIG_FRAC / NS) // 2 * 2
    small_total = total_chunks - NS * nch_big
    nch_small = small_total // NS // 2 * 2
    extra2 = (small_total - NS * nch_small) // 2
    assert nch_small * NS + 2 * extra2 == small_total and extra2 <= NS
    stage_max = max(nch_big, nch_small + (2 if extra2 else 0)) * CW

    mesh = plsc.VectorSubcoreMesh(core_axis_name="c", subcore_axis_name="s")
    assert (NW * msg_dim) % 128 == 0

    @functools.partial(
        pl.kernel,
        mesh=mesh,
        out_type=jax.ShapeDtypeStruct((NW * msg_dim // 128, 128), jnp.float32),
        compiler_params=pltpu.CompilerParams(use_tc_tiling_on_sc=False),
        scratch_types=[
            pltpu.VMEM((stage_max,), jnp.int32),
            pltpu.VMEM((stage_max,), jnp.int32),
            pltpu.VMEM((CW, msg_dim), jnp.float32),
            pltpu.VMEM((CW, msg_dim), jnp.float32),
            pltpu.VMEM((CW, msg_dim), jnp.float32),
            pltpu.VMEM((CW, msg_dim), jnp.float32),
            pltpu.VMEM((msg_dim,), jnp.float32),
            pltpu.SemaphoreType.DMA,
            pltpu.SemaphoreType.DMA,
            pltpu.SemaphoreType.DMA,
            pltpu.SemaphoreType.DMA,
        ],
    )
    def edge_l1(t_hbm, ei_hbm, out_hbm,
                sidx, didx, pbuf0, qbuf0, pbuf1, qbuf1, accv,
                sem_p0, sem_q0, sem_p1, sem_q1):
        c = lax.axis_index("c")
        s = lax.axis_index("s")
        wid = s * NC + c
        is_big = c == BIG_CORE
        nc_mine = jnp.where(is_big, nch_big,
                            jnp.where(s < extra2, nch_small + 2, nch_small))
        start_chunk = jnp.where(
            is_big, small_total + s * nch_big,
            s * nch_small + 2 * jnp.minimum(s, extra2))
        e0 = start_chunk * CW
        stage_start = jnp.minimum(e0, n_edges - stage_max)
        off = e0 - stage_start
        pltpu.sync_copy(ei_hbm.at[0, pl.ds(stage_start, stage_max)], sidx)
        pltpu.sync_copy(ei_hbm.at[1, pl.ds(stage_start, stage_max)], didx)

        def issue(k, pb, qb, sp, sq):
            for j in range(CW // 16):
                sl = pl.ds(off + k * CW + j * 16, 16)
                sidx[sl] = sidx[sl] * 2
                didx[sl] = didx[sl] * 2 + 1
            pltpu.async_copy(t_hbm.at[sidx.at[pl.ds(off + k * CW, CW)]], pb, sp)
            pltpu.async_copy(t_hbm.at[didx.at[pl.ds(off + k * CW, CW)]], qb, sq)

        def drain(k, pb, qb, sp, sq):
            pltpu.make_async_copy(t_hbm.at[sidx.at[pl.ds(off + k * CW, CW)]], pb, sp).wait()
            pltpu.make_async_copy(t_hbm.at[didx.at[pl.ds(off + k * CW, CW)]], qb, sq).wait()

        def accum(pb, qb, acc):
            def lane_body(i, carry):
                a0, a1 = carry
                j = i * 2
                a0 = a0 + jnp.abs(pb[j] + qb[j])
                a1 = a1 + jnp.abs(pb[j + 1] + qb[j + 1])
                return a0, a1

            return lax.fori_loop(0, CW // 2, lane_body, acc, unroll=4)

        issue(0, pbuf0, qbuf0, sem_p0, sem_q0)
        zero = jnp.zeros((msg_dim,), jnp.float32)

        def pair_body(h, acc):
            k = h * 2
            issue(k + 1, pbuf1, qbuf1, sem_p1, sem_q1)
            drain(k, pbuf0, qbuf0, sem_p0, sem_q0)
            acc = accum(pbuf0, qbuf0, acc)

            @pl.when(k + 2 < nc_mine)
            def _():
                issue(k + 2, pbuf0, qbuf0, sem_p0, sem_q0)

            drain(k + 1, pbuf1, qbuf1, sem_p1, sem_q1)
            return accum(pbuf1, qbuf1, acc)

        a0, a1 = lax.fori_loop(0, nc_mine // 2, pair_body, (zero, zero))
        accv[...] = a0 + a1
        per_row = 128 // msg_dim
        pltpu.sync_copy(
            accv, out_hbm.at[wid // per_row, pl.ds((wid % per_row) * msg_dim, msg_dim)])

    return edge_l1


def kernel(y, target, x, edge_index, W_msg, b_msg):
    n_nodes, d_feat = x.shape
    n_edges = edge_index.shape[1]
    msg_dim = W_msg.shape[1]

    ei = edge_index.astype(jnp.int32)
    b2 = b_msg.reshape(1, msg_dim)

    tables = pl.pallas_call(
        _tables_body,
        out_shape=jax.ShapeDtypeStruct((n_nodes, 2 * msg_dim), jnp.float32),
    )
    pq = tables(x, W_msg, b2)
    tab = pq.reshape(2 * n_nodes, msg_dim)

    partials = _make_edge_l1(n_edges, msg_dim)(tab, ei)

    y2 = y.reshape(80, -1)
    t2 = target.reshape(80, -1)
    combine = pl.pallas_call(
        functools.partial(_combine_body, n_nodes, n_edges),
        out_shape=jax.ShapeDtypeStruct((1, 1), jnp.float32),
    )
    return combine(y2, t2, partials)[0, 0]


# speedup vs baseline: 1.0952x; 1.0193x over previous
import functools

import jax
import jax.numpy as jnp
from jax import lax
from jax.experimental import pallas as pl
from jax.experimental.pallas import tpu as pltpu
from jax.experimental.pallas import tpu_sc as plsc

REG_WEIGHT_ = 0.01
NC = 2
NS = 16
NW = NC * NS
CW = 128
BIG_CORE = 0
BIG_FRAC = 0.507


def _tables_body(x_ref, w_ref, b_ref, pq_ref):
    d = x_ref.shape[1]
    x = x_ref[...]
    p = jnp.dot(x, w_ref[:d, :], preferred_element_type=jnp.float32) + b_ref[...]
    q = jnp.dot(x, w_ref[d:, :], preferred_element_type=jnp.float32)
    pq_ref[...] = jnp.concatenate([p, q], axis=1)


def _combine_body(n_nodes, n_edges, y_ref, t_ref, part_ref, o_ref):
    base = jnp.sum(jnp.abs(y_ref[...] - t_ref[...]))
    l1 = jnp.sum(part_ref[...])
    total = base / n_nodes + REG_WEIGHT_ * (l1 / n_edges)
    o_ref[...] = jnp.reshape(total, (1, 1))


def _make_edge_l1(n_edges, msg_dim):
    total_chunks = n_edges // CW
    nch_big = int(total_chunks * BIG_FRAC / NS) // 2 * 2
    small_total = total_chunks - NS * nch_big
    nch_small = small_total // NS // 2 * 2
    extra2 = (small_total - NS * nch_small) // 2
    assert nch_small * NS + 2 * extra2 == small_total and extra2 <= NS
    stage_max = max(nch_big, nch_small + (2 if extra2 else 0)) * CW

    mesh = plsc.VectorSubcoreMesh(core_axis_name="c", subcore_axis_name="s")
    assert (NW * msg_dim) % 128 == 0

    @functools.partial(
        pl.kernel,
        mesh=mesh,
        out_type=jax.ShapeDtypeStruct((NW * msg_dim // 128, 128), jnp.float32),
        compiler_params=pltpu.CompilerParams(use_tc_tiling_on_sc=False),
        scratch_types=[
            pltpu.VMEM((stage_max,), jnp.int32),
            pltpu.VMEM((stage_max,), jnp.int32),
            pltpu.VMEM((CW, msg_dim), jnp.float32),
            pltpu.VMEM((CW, msg_dim), jnp.float32),
            pltpu.VMEM((CW, msg_dim), jnp.float32),
            pltpu.VMEM((CW, msg_dim), jnp.float32),
            pltpu.VMEM((msg_dim,), jnp.float32),
            pltpu.SemaphoreType.DMA,
            pltpu.SemaphoreType.DMA,
            pltpu.SemaphoreType.DMA,
            pltpu.SemaphoreType.DMA,
        ],
    )
    def edge_l1(t_hbm, ei_hbm, out_hbm,
                sidx, didx, pbuf0, qbuf0, pbuf1, qbuf1, accv,
                sem_p0, sem_q0, sem_p1, sem_q1):
        c = lax.axis_index("c")
        s = lax.axis_index("s")
        wid = s * NC + c
        is_big = c == BIG_CORE
        nc_mine = jnp.where(is_big, nch_big,
                            jnp.where(s < extra2, nch_small + 2, nch_small))
        start_chunk = jnp.where(
            is_big, small_total + s * nch_big,
            s * nch_small + 2 * jnp.minimum(s, extra2))
        e0 = start_chunk * CW
        stage_start = jnp.minimum(e0, n_edges - stage_max)
        off = e0 - stage_start
        pltpu.sync_copy(ei_hbm.at[0, pl.ds(stage_start, stage_max)], sidx)
        pltpu.sync_copy(ei_hbm.at[1, pl.ds(stage_start, stage_max)], didx)

        def issue(k, pb, qb, sp, sq):
            pltpu.async_copy(t_hbm.at[sidx.at[pl.ds(off + k * CW, CW)]], pb, sp)
            pltpu.async_copy(t_hbm.at[didx.at[pl.ds(off + k * CW, CW)]], qb, sq)

        def drain(k, pb, qb, sp, sq):
            pltpu.make_async_copy(t_hbm.at[sidx.at[pl.ds(off + k * CW, CW)]], pb, sp).wait()
            pltpu.make_async_copy(t_hbm.at[didx.at[pl.ds(off + k * CW, CW)]], qb, sq).wait()

        def accum(pb, qb, acc):
            def lane_body(i, carry):
                a0, a1 = carry
                j = i * 2
                a0 = a0 + jnp.abs(pb[j] + qb[j])
                a1 = a1 + jnp.abs(pb[j + 1] + qb[j + 1])
                return a0, a1

            return lax.fori_loop(0, CW // 2, lane_body, acc, unroll=4)

        issue(0, pbuf0, qbuf0, sem_p0, sem_q0)
        zero = jnp.zeros((msg_dim,), jnp.float32)

        def pair_body(h, acc):
            k = h * 2
            issue(k + 1, pbuf1, qbuf1, sem_p1, sem_q1)
            drain(k, pbuf0, qbuf0, sem_p0, sem_q0)
            acc = accum(pbuf0, qbuf0, acc)

            @pl.when(k + 2 < nc_mine)
            def _():
                issue(k + 2, pbuf0, qbuf0, sem_p0, sem_q0)

            drain(k + 1, pbuf1, qbuf1, sem_p1, sem_q1)
            return accum(pbuf1, qbuf1, acc)

        a0, a1 = lax.fori_loop(0, nc_mine // 2, pair_body, (zero, zero))
        accv[...] = a0 + a1
        per_row = 128 // msg_dim
        pltpu.sync_copy(
            accv, out_hbm.at[wid // per_row, pl.ds((wid % per_row) * msg_dim, msg_dim)])

    return edge_l1


def kernel(y, target, x, edge_index, W_msg, b_msg):
    n_nodes, d_feat = x.shape
    n_edges = edge_index.shape[1]
    msg_dim = W_msg.shape[1]

    ei = edge_index.astype(jnp.int32)
    ei = ei * 2 + jnp.array([[0], [1]], jnp.int32)
    b2 = b_msg.reshape(1, msg_dim)

    tables = pl.pallas_call(
        _tables_body,
        out_shape=jax.ShapeDtypeStruct((n_nodes, 2 * msg_dim), jnp.float32),
    )
    pq = tables(x, W_msg, b2)
    tab = pq.reshape(2 * n_nodes, msg_dim)

    partials = _make_edge_l1(n_edges, msg_dim)(tab, ei)

    y2 = y.reshape(80, -1)
    t2 = target.reshape(80, -1)
    combine = pl.pallas_call(
        functools.partial(_combine_body, n_nodes, n_edges),
        out_shape=jax.ShapeDtypeStruct((1, 1), jnp.float32),
    )
    return combine(y2, t2, partials)[0, 0]
